# baseline (speedup 1.0000x reference)
import functools

import jax
import jax.numpy as jnp
from jax import lax
from jax.experimental import pallas as pl
from jax.experimental.pallas import tpu as pltpu
from jax.experimental.pallas import tpu_sc as plsc

N = 16384
D = 64
FD = 128
K = 16


@functools.lru_cache(maxsize=1)
def _make_gather():
    info = plsc.get_sparse_core_info()
    nc, ns = info.num_cores, info.num_subcores
    nw = nc * ns
    bpw = N // nw
    mesh = plsc.VectorSubcoreMesh(core_axis_name="c", subcore_axis_name="s")

    @functools.partial(
        pl.kernel,
        mesh=mesh,
        out_type=jax.ShapeDtypeStruct((N, 8, D), jnp.float32),
        scratch_types=[
            pltpu.VMEM((bpw,), jnp.int32),
            pltpu.SemaphoreType.DMA,
        ],
    )
    def gather_k(table_hbm, idx_hbm, out_hbm, idx_s, sem):
        wid = lax.axis_index("s") * nc + lax.axis_index("c")
        base = wid * bpw
        pltpu.sync_copy(idx_hbm.at[wid], idx_s)

        @pl.loop(0, bpw // K)
        def _(o):
            lo = o * K
            v = idx_s[pl.ds(lo, K)]
            handles = [
                pltpu.async_copy(
                    table_hbm.at[v[j]],
                    out_hbm.at[base + lo + j],
                    sem,
                )
                for j in range(K)
            ]
            for h in handles:
                h.wait()

    return gather_k, nw, bpw


BLK = 2048


def _proj_body(ff_ref, w_ref, b_ref, slab_ref, r_ref, o_ref):
    sub = jax.lax.broadcasted_iota(jnp.int32, (BLK, 8, D), 1)
    onehot = (sub == r_ref[...].reshape(BLK, 1, 1)).astype(jnp.float32)
    sel = jnp.sum(slab_ref[...] * onehot, axis=1)
    o_ref[...] = (
        jnp.dot(ff_ref[...], w_ref[...], preferred_element_type=jnp.float32)
        + b_ref[...]
        + sel
    )


def kernel(float_feat, id_feat, W, b, emb_table):
    gather_k, nw, bpw = _make_gather()
    idx = id_feat.astype(jnp.int32)
    slab_idx = (idx >> 3).reshape(nw, bpw)
    row_in_slab = (idx & 7).reshape(N, 1)
    table3 = emb_table.reshape(125000, 8, D)
    slabs = gather_k(table3, slab_idx)
    return pl.pallas_call(
        _proj_body,
        grid=(N // BLK,),
        in_specs=[
            pl.BlockSpec((BLK, FD), lambda i: (i, 0)),
            pl.BlockSpec((FD, D), lambda i: (0, 0)),
            pl.BlockSpec((1, D), lambda i: (0, 0)),
            pl.BlockSpec((BLK, 8, D), lambda i: (i, 0, 0)),
            pl.BlockSpec((BLK, 1), lambda i: (i, 0)),
        ],
        out_specs=pl.BlockSpec((BLK, D), lambda i: (i, 0)),
        out_shape=jax.ShapeDtypeStruct((N, D), jnp.float32),
    )(float_feat, W, b.reshape(1, D), slabs, row_in_slab)


# speedup vs baseline: 6.8692x; 6.8692x over previous
import functools

import jax
import jax.numpy as jnp
from jax import lax
from jax.experimental import pallas as pl
from jax.experimental.pallas import tpu as pltpu
from jax.experimental.pallas import tpu_sc as plsc

N = 16384
D = 64
FD = 128
K = 32


@functools.lru_cache(maxsize=1)
def _make_gather():
    info = plsc.get_sparse_core_info()
    nc, ns = info.num_cores, info.num_subcores
    nw = nc * ns
    bpw = N // nw
    mesh = plsc.VectorSubcoreMesh(core_axis_name="c", subcore_axis_name="s")

    @functools.partial(
        pl.kernel,
        mesh=mesh,
        out_type=jax.ShapeDtypeStruct((N, 8, D), jnp.float32),
        scratch_types=[
            pltpu.VMEM((bpw,), jnp.int32),
            pltpu.VMEM((K, 8, D), jnp.float32),
            pltpu.VMEM((K, 8, D), jnp.float32),
            pltpu.SemaphoreType.DMA,
            pltpu.SemaphoreType.DMA,
        ],
    )
    def gather_k(table_hbm, idx_hbm, out_hbm, idx_s, buf0, buf1, gsem, osem):
        wid = lax.axis_index("s") * nc + lax.axis_index("c")
        base = wid * bpw
        pltpu.sync_copy(idx_hbm.at[wid], idx_s)
        bufs = (buf0, buf1)
        nrounds = bpw // K
        ghandles = [None, None]
        ohandles = [None, None]
        for o in range(nrounds):
            p = o & 1
            if ohandles[p] is not None:
                ohandles[p].wait()
            lo = o * K
            hs = []
            for jlo in range(0, K, 16):
                v = idx_s[pl.ds(lo + jlo, 16)]
                for j in range(16):
                    hs.append(pltpu.async_copy(
                        table_hbm.at[v[j]],
                        bufs[p].at[jlo + j],
                        gsem,
                    ))
            ghandles[p] = hs
            q = 1 - p
            if ghandles[q] is not None:
                for h in ghandles[q]:
                    h.wait()
                ohandles[q] = pltpu.async_copy(
                    bufs[q],
                    out_hbm.at[pl.ds(base + (o - 1) * K, K)],
                    osem,
                )
                ghandles[q] = None
        last = (nrounds - 1) & 1
        for h in ghandles[last]:
            h.wait()
        pltpu.sync_copy(bufs[last], out_hbm.at[pl.ds(base + (nrounds - 1) * K, K)])
        if ohandles[1 - last] is not None:
            ohandles[1 - last].wait()

    return gather_k, nw, bpw


BLK = 2048


def _proj_body(ff_ref, w_ref, b_ref, slab_ref, r_ref, o_ref):
    sub = jax.lax.broadcasted_iota(jnp.int32, (BLK, 8, D), 1)
    onehot = (sub == r_ref[...].reshape(BLK, 1, 1)).astype(jnp.float32)
    sel = jnp.sum(slab_ref[...] * onehot, axis=1)
    o_ref[...] = (
        jnp.dot(ff_ref[...], w_ref[...], preferred_element_type=jnp.float32)
        + b_ref[...]
        + sel
    )


def kernel(float_feat, id_feat, W, b, emb_table):
    gather_k, nw, bpw = _make_gather()
    idx = id_feat.astype(jnp.int32)
    slab_idx = (idx >> 3).reshape(nw, bpw)
    row_in_slab = (idx & 7).reshape(N, 1)
    table3 = emb_table.reshape(125000, 8, D)
    slabs = gather_k(table3, slab_idx)
    return pl.pallas_call(
        _proj_body,
        grid=(N // BLK,),
        in_specs=[
            pl.BlockSpec((BLK, FD), lambda i: (i, 0)),
            pl.BlockSpec((FD, D), lambda i: (0, 0)),
            pl.BlockSpec((1, D), lambda i: (0, 0)),
            pl.BlockSpec((BLK, 8, D), lambda i: (i, 0, 0)),
            pl.BlockSpec((BLK, 1), lambda i: (i, 0)),
        ],
        out_specs=pl.BlockSpec((BLK, D), lambda i: (i, 0)),
        out_shape=jax.ShapeDtypeStruct((N, D), jnp.float32),
    )(float_feat, W, b.reshape(1, D), slabs, row_in_slab)
